# baseline (speedup 1.0000x reference)
import functools

import jax
import jax.numpy as jnp
import numpy as np
from jax import lax
from jax.experimental import pallas as pl
from jax.experimental.pallas import tpu as pltpu
from jax.experimental.pallas import tpu_sc as plsc

NC = 2
NS = 16
L = 16
NW = NC * NS


def _col_perm(d):
    perm = np.empty((d,), np.int32)
    for k in range(d // 32):
        for i in range(16):
            perm[32 * k + 2 * i] = 32 * k + i
            perm[32 * k + 2 * i + 1] = 32 * k + 16 + i
    return perm


def _matmul_bf16(x, Wp):
    n, d_in = x.shape
    d_out = Wp.shape[1]
    bm = 1000

    def body(x_ref, w_ref, o_ref):
        r = jnp.dot(x_ref[...], w_ref[...], preferred_element_type=jnp.float32)
        o_ref[...] = r.astype(jnp.bfloat16)

    return pl.pallas_call(
        body,
        grid=(n // bm,),
        in_specs=[
            pl.BlockSpec((bm, d_in), lambda i: (i, 0)),
            pl.BlockSpec((d_in, d_out), lambda i: (0, 0)),
        ],
        out_specs=pl.BlockSpec((bm, d_out), lambda i: (i, 0)),
        out_shape=jax.ShapeDtypeStruct((n, d_out), jnp.bfloat16),
    )(x, Wp)


def _sc_scatter(sup, ei, ew):
    n, dw = sup.shape
    d = dw * 2
    e = ew.shape[0]
    epw = e // NW
    c = 80
    sck = 25
    nsc = epw // (sck * c)
    rblk = 80
    nblk = n // rblk
    dvec = d // L

    ei4 = ei.reshape(2, NW * nsc, sck, c)
    ew2 = ew.reshape(NW * nsc, sck * c)

    mesh = plsc.VectorSubcoreMesh(core_axis_name="c", subcore_axis_name="s")

    @functools.partial(
        pl.kernel,
        out_type=jax.ShapeDtypeStruct((NC, n, d), jnp.float32),
        mesh=mesh,
        compiler_params=pltpu.CompilerParams(
            needs_layout_passes=False, use_tc_tiling_on_sc=False),
        scratch_types=[
            pltpu.VMEM((sck, c), jnp.int32),
            pltpu.VMEM((sck, c), jnp.int32),
            pltpu.VMEM((sck * c,), jnp.float32),
            pltpu.VMEM((c, dw), jnp.int32),
            pltpu.VMEM((c, dw), jnp.int32),
            pltpu.VMEM((c, dw), jnp.int32),
            pltpu.VMEM((c, d), jnp.float32),
            pltpu.VMEM((c, d), jnp.float32),
            pltpu.VMEM_SHARED((n, d), jnp.float32),
            pltpu.SemaphoreType.DMA,
            pltpu.SemaphoreType.DMA,
            pltpu.SemaphoreType.DMA,
            pltpu.SemaphoreType.DMA,
            pltpu.SemaphoreType.DMA,
        ],
    )
    def sc_body(sup_hbm, ei_hbm, ew_hbm, out_hbm,
                src_v, dst_v, ew_v, ib0, ib1, ib2, fb0, fb1, acc,
                gsem0, gsem1, gsem2, ssem0, ssem1):
        cid = lax.axis_index("c")
        sid = lax.axis_index("s")
        wid = cid * NS + sid
        nb = (nblk - 1 - sid) // NS + 1

        zv = jnp.zeros((L,), jnp.float32)

        def zrow(i, _):
            for j in range(dvec):
                fb0[i, pl.ds(j * L, L)] = zv
            return 0

        lax.fori_loop(0, c, zrow, 0)

        def zblk(k, _):
            r = pl.multiple_of((sid + k * NS) * rblk, 8)
            pltpu.sync_copy(fb0, acc.at[pl.ds(r, rblk)])
            return 0

        lax.fori_loop(0, nb, zblk, 0)
        plsc.subcore_barrier()

        ibufs = ((ib0, gsem0), (ib1, gsem1), (ib2, gsem2))
        fbufs = ((fb0, ssem0), (fb1, ssem1))

        def scale(ib, fb, i):
            def sgroup(g, _):
                ew16 = ew_v[pl.ds(i * c + g * L, L)]
                for t in range(L):
                    wb = ew16.at[jnp.full((L,), t, jnp.int32)].get(
                        mode="promise_in_bounds")
                    row = g * L + t
                    for k in range(dw // L):
                        w32 = ib[row, pl.ds(k * L, L)]
                        a = lax.bitcast_convert_type(w32 << 16, jnp.float32)
                        b = lax.bitcast_convert_type(
                            w32 & jnp.int32(-65536), jnp.float32)
                        fb[row, pl.ds(2 * k * L, L)] = a * wb
                        fb[row, pl.ds((2 * k + 1) * L, L)] = b * wb
                return 0

            lax.fori_loop(0, c // L, sgroup, 0)

        def gather(i, ib, sem):
            pltpu.async_copy(sup_hbm.at[src_v.at[i]], ib, sem)

        def gwait(i, ib, sem):
            pltpu.make_async_copy(sup_hbm.at[src_v.at[i]], ib, sem).wait()

        def scatter_start(fb, i, sem):
            pltpu.async_copy(fb, acc.at[dst_v.at[i]], sem, add=True)

        def scatter_drain(fb, i, sem):
            pltpu.make_async_copy(fb, acc.at[dst_v.at[i]], sem).wait()

        def superchunk(s, _):
            sc_row = wid * nsc + s
            pltpu.sync_copy(ei_hbm.at[0, sc_row], src_v)
            pltpu.sync_copy(ei_hbm.at[1, sc_row], dst_v)
            pltpu.sync_copy(ew_hbm.at[sc_row], ew_v)

            gather(0, ib0, gsem0)
            gather(1, ib1, gsem1)

            def sextet(t, _):
                for p in range(6):
                    i = t * 6 + p
                    ib, gs = ibufs[p % 3]
                    fb, ss = fbufs[p % 2]
                    nib, ngs = ibufs[(p + 2) % 3]
                    gwait(i, ib, gs)
                    if p == 5:
                        @pl.when(t < (sck // 6) - 1)
                        def _():
                            gather(i + 2, nib, ngs)
                    else:
                        gather(i + 2, nib, ngs)
                    if p < 2:
                        @pl.when(t >= 1)
                        def _():
                            scatter_drain(fb, i - 2, ss)
                    else:
                        scatter_drain(fb, i - 2, ss)
                    scale(ib, fb, i)
                    scatter_start(fb, i, ss)
                return 0

            lax.fori_loop(0, sck // 6, sextet, 0)
            last = sck - 1
            gwait(last, ib0, gsem0)
            scatter_drain(fb0, last - 2, ssem0)
            scale(ib0, fb0, last)
            pltpu.sync_copy(fb0, acc.at[dst_v.at[last]], add=True)
            scatter_drain(fb1, last - 1, ssem1)
            return 0

        lax.fori_loop(0, nsc, superchunk, 0)
        plsc.subcore_barrier()

        def wblk(k, _):
            r = pl.multiple_of((sid + k * NS) * rblk, 8)
            pltpu.sync_copy(acc.at[pl.ds(r, rblk)],
                            out_hbm.at[cid].at[pl.ds(r, rblk)])
            return 0

        lax.fori_loop(0, nb, wblk, 0)

    return sc_body(sup, ei4, ew2)


def _combine(partials):
    _, n, d = partials.shape
    bm = 1000

    def body(p_ref, o_ref):
        o_ref[...] = jnp.maximum(p_ref[0] + p_ref[1], 0.0)

    return pl.pallas_call(
        body,
        grid=(n // bm,),
        in_specs=[pl.BlockSpec((NC, bm, d), lambda i: (0, i, 0))],
        out_specs=pl.BlockSpec((bm, d), lambda i: (i, 0)),
        out_shape=jax.ShapeDtypeStruct((n, d), jnp.float32),
    )(partials)


@jax.jit
def kernel(x, edge_index, edge_weight, W):
    d = W.shape[1]
    Wp = W[:, _col_perm(d)]
    sup_bf = _matmul_bf16(x, Wp)
    n = sup_bf.shape[0]
    sup32 = lax.bitcast_convert_type(
        sup_bf.reshape(n, d // 2, 2), jnp.int32)
    partials = _sc_scatter(sup32, edge_index, edge_weight)
    return _combine(partials)
